# baseline (speedup 1.0000x reference)
import functools

import jax
import jax.numpy as jnp
from jax import lax
from jax.experimental import pallas as pl
from jax.experimental.pallas import tpu as pltpu
from jax.experimental.pallas import tpu_sc as plsc

N = 10000
E = 320000
D = 128
DH = D // 2

NSUB = 16
ROWS_J = 128
NJB = 16
NT = 10
NJ = NJB * NT
EPT = NJ * ROWS_J
EPAD = NSUB * EPT
NP = 10240
NQ = NP // 2
RPT = NQ // NSUB

BN = 2000
BQ = BN // 2


def _dot_wt(h, w):
    return lax.dot_general(h, w, (((1,), (1,)), ((), ())),
                           precision=lax.Precision.HIGHEST,
                           preferred_element_type=jnp.float32)


def _split_store(hh_ref, hh):
    hhp = hh.reshape(BQ, 2, D)
    ev = hhp[:, 0, :]
    od = hhp[:, 1, :]
    hh_ref[0] = jnp.concatenate([ev[:, :DH], od[:, :DH]], axis=-1)
    hh_ref[1] = jnp.concatenate([ev[:, DH:], od[:, DH:]], axis=-1)


def _merge(s_ref):
    s0 = s_ref[0]
    s1 = s_ref[1]
    ev = jnp.concatenate([s0[:, :DH], s1[:, :DH]], axis=-1)
    od = jnp.concatenate([s0[:, DH:], s1[:, DH:]], axis=-1)
    return jnp.stack([ev, od], axis=1).reshape(BN, D)


def _tc_first_body(temp_ref, x_ref, w_ref, b_ref, hh_ref, hid_ref):
    xb = x_ref[...]
    _split_store(hh_ref, _dot_wt(xb, w_ref[...]) + b_ref[...])
    hid_ref[...] = xb * temp_ref[0]


def _tc_mid_body(temp_ref, s_ref, hid_ref, w_ref, b_ref, hh_ref, hidout_ref,
                 *, layer):
    h = jnp.maximum(_merge(s_ref), 0.0)
    hidout_ref[...] = hid_ref[...] + h * temp_ref[layer]
    _split_store(hh_ref, _dot_wt(h, w_ref[...]) + b_ref[...])


def _tc_last_body(temp_ref, s_ref, hid_ref, out_ref):
    h = jnp.maximum(_merge(s_ref), 0.0)
    out_ref[...] = hid_ref[...] + h * temp_ref[3]


_SPEC_T = pl.BlockSpec(memory_space=pltpu.SMEM)
_SPEC_X = pl.BlockSpec((BN, D), lambda i: (i, 0))
_SPEC_S = pl.BlockSpec((2, BQ, D), lambda i: (0, i, 0))
_SPEC_W = pl.BlockSpec((D, D), lambda i: (0, 0))
_SPEC_B = pl.BlockSpec((1, D), lambda i: (0, 0))

_GRID = N // BN


def _tc_first(temp, x, w, b):
    return pl.pallas_call(
        _tc_first_body,
        grid=(_GRID,),
        in_specs=[_SPEC_T, _SPEC_X, _SPEC_W, _SPEC_B],
        out_specs=[_SPEC_S, _SPEC_X],
        out_shape=[jax.ShapeDtypeStruct((2, NQ, D), jnp.float32),
                   jax.ShapeDtypeStruct((N, D), jnp.float32)],
    )(temp, x, w, b)


def _tc_mid(temp, s, hid, w, b, layer):
    return pl.pallas_call(
        functools.partial(_tc_mid_body, layer=layer),
        grid=(_GRID,),
        in_specs=[_SPEC_T, _SPEC_S, _SPEC_X, _SPEC_W, _SPEC_B],
        out_specs=[_SPEC_S, _SPEC_X],
        out_shape=[jax.ShapeDtypeStruct((2, NQ, D), jnp.float32),
                   jax.ShapeDtypeStruct((N, D), jnp.float32)],
    )(temp, s, hid, w, b)


def _tc_last(temp, s, hid):
    return pl.pallas_call(
        _tc_last_body,
        grid=(_GRID,),
        in_specs=[_SPEC_T, _SPEC_S, _SPEC_X],
        out_specs=_SPEC_X,
        out_shape=jax.ShapeDtypeStruct((N, D), jnp.float32),
    )(temp, s, hid)


def _sc_body(hh_hbm, srcq_hbm, soff_hbm, dstq_hbm, doff_hbm, ew_hbm, out_hbm,
             hh_sh, acc_sh, srcq_v, soff_v, dstq_v, doff_v, ew_v,
             rows_a, rows_b, gsem_a, gsem_b, ssem_a, ssem_b):
    c = lax.axis_index("c")
    s = lax.axis_index("s")

    def _zrow(r, carry):
        for k in range(D // 16):
            rows_a[r, pl.ds(k * 16, 16)] = jnp.zeros((16,), jnp.float32)
        return carry
    lax.fori_loop(0, ROWS_J, _zrow, 0)
    pltpu.sync_copy(rows_a, acc_sh.at[pl.ds(s * RPT, ROWS_J)])
    pltpu.sync_copy(rows_a, acc_sh.at[pl.ds(s * RPT + ROWS_J, ROWS_J)])
    pltpu.sync_copy(rows_a.at[pl.ds(0, RPT - 2 * ROWS_J)],
                    acc_sh.at[pl.ds(s * RPT + 2 * ROWS_J, RPT - 2 * ROWS_J)])

    pltpu.sync_copy(hh_hbm.at[c, pl.ds(s * RPT, RPT)],
                    hh_sh.at[pl.ds(s * RPT, RPT)])

    plsc.subcore_barrier()

    def _scale(rows, j):
        def _scale16(g, carry2):
            wv = ew_v[j, pl.ds(g * 16, 16)]
            sv = soff_v[j, pl.ds(g * 16, 16)]
            dv = doff_v[j, pl.ds(g * 16, 16)]
            zero = jnp.zeros((16,), jnp.float32)
            for m in range(16):
                e = g * 16 + m
                ew = wv[m]
                so = sv[m]
                do = dv[m]
                vals = [rows[e, pl.ds(so + k * 16, 16)] * ew
                        for k in range(DH // 16)]
                for k in range(DH // 16):
                    rows[e, pl.ds(do + k * 16, 16)] = vals[k]
                    rows[e, pl.ds((DH - do) + k * 16, 16)] = zero
            return carry2
        lax.fori_loop(0, ROWS_J // 16, _scale16, 0)

    bufs = ((rows_a, gsem_a, ssem_a), (rows_b, gsem_b, ssem_b))

    def _gather(j, buf, gsem):
        pltpu.async_copy(hh_sh.at[srcq_v.at[j]], buf, gsem)

    def _superchunk(t, carry):
        pltpu.sync_copy(srcq_hbm.at[s, pl.ds(t * NJB, NJB)], srcq_v)
        pltpu.sync_copy(soff_hbm.at[s, pl.ds(t * NJB, NJB)], soff_v)
        pltpu.sync_copy(dstq_hbm.at[s, pl.ds(t * NJB, NJB)], dstq_v)
        pltpu.sync_copy(doff_hbm.at[s, pl.ds(t * NJB, NJB)], doff_v)
        pltpu.sync_copy(ew_hbm.at[s, pl.ds(t * NJB, NJB)], ew_v)

        _gather(0, rows_a, gsem_a)
        _gather(1, rows_b, gsem_b)

        def _pair(i, carry1):
            for b, (buf, gsem, ssem) in enumerate(bufs):
                j = 2 * i + b
                pltpu.make_async_copy(hh_sh.at[srcq_v.at[j]], buf, gsem).wait()
                _scale(buf, j)
                pltpu.async_copy(buf, acc_sh.at[dstq_v.at[j]], ssem, add=True)

                @pl.when(i < NJB // 2 - 1)
                def _prefetch():
                    pltpu.make_async_copy(
                        buf, acc_sh.at[dstq_v.at[j]], ssem).wait()
                    _gather(j + 2, buf, gsem)
            return carry1
        lax.fori_loop(0, NJB // 2, _pair, 0)

        for b, (buf, gsem, ssem) in enumerate(bufs):
            j = NJB - 2 + b
            pltpu.make_async_copy(buf, acc_sh.at[dstq_v.at[j]], ssem).wait()
        return carry
    lax.fori_loop(0, NT, _superchunk, 0)

    plsc.subcore_barrier()

    pltpu.sync_copy(acc_sh.at[pl.ds(s * RPT, RPT)],
                    out_hbm.at[c, pl.ds(s * RPT, RPT)])


@functools.lru_cache(maxsize=1)
def _make_sc_propagate():
    return functools.partial(
        pl.kernel,
        out_type=jax.ShapeDtypeStruct((2, NQ, D), jnp.float32),
        mesh=plsc.VectorSubcoreMesh(core_axis_name="c", subcore_axis_name="s",
                                    num_cores=2, num_subcores=NSUB),
        scratch_types=[
            pltpu.VMEM_SHARED((NQ, D), jnp.float32),
            pltpu.VMEM_SHARED((NQ, D), jnp.float32),
            pltpu.VMEM((NJB, ROWS_J), jnp.int32),
            pltpu.VMEM((NJB, ROWS_J), jnp.int32),
            pltpu.VMEM((NJB, ROWS_J), jnp.int32),
            pltpu.VMEM((NJB, ROWS_J), jnp.int32),
            pltpu.VMEM((NJB, ROWS_J), jnp.float32),
            pltpu.VMEM((ROWS_J, D), jnp.float32),
            pltpu.VMEM((ROWS_J, D), jnp.float32),
            pltpu.SemaphoreType.DMA,
            pltpu.SemaphoreType.DMA,
            pltpu.SemaphoreType.DMA,
            pltpu.SemaphoreType.DMA,
        ],
    )(_sc_body)


def _sc_propagate(*args):
    return _make_sc_propagate()(*args)


def kernel(x, edge_index, edge_weight, temp, W0, b0, W1, b1, W2, b2):
    pad = EPAD - E
    src = jnp.pad(edge_index[0].astype(jnp.int32), (0, pad))
    dst = jnp.pad(edge_index[1].astype(jnp.int32), (0, pad))
    ew = jnp.pad(edge_weight, (0, pad))
    srcq_t = (src // 2).reshape(NSUB, NJ, ROWS_J)
    soff_t = ((src % 2) * DH).reshape(NSUB, NJ, ROWS_J)
    dstq_t = (dst // 2).reshape(NSUB, NJ, ROWS_J)
    doff_t = ((dst % 2) * DH).reshape(NSUB, NJ, ROWS_J)
    ew_t = ew.reshape(NSUB, NJ, ROWS_J)
    edges = (srcq_t, soff_t, dstq_t, doff_t, ew_t)

    hh, hidden = _tc_first(temp, x, W0, b0.reshape(1, D))
    s1 = _sc_propagate(hh, *edges)
    hh, hidden = _tc_mid(temp, s1, hidden, W1, b1.reshape(1, D), 1)
    s2 = _sc_propagate(hh, *edges)
    hh, hidden = _tc_mid(temp, s2, hidden, W2, b2.reshape(1, D), 2)
    s3 = _sc_propagate(hh, *edges)
    return _tc_last(temp, s3, hidden)


# speedup vs baseline: 1.3535x; 1.3535x over previous
import functools

import jax
import jax.numpy as jnp
from jax import lax
from jax.experimental import pallas as pl
from jax.experimental.pallas import tpu as pltpu
from jax.experimental.pallas import tpu_sc as plsc

N = 10000
E = 320000
D = 128

NTILE = 32
NSUB = 16
ROWS_J = 128
NJB = 16
NT = 5
NJ = NJB * NT
EPT = NJ * ROWS_J
EPAD = NTILE * EPT
NP = 10240
RPT = NP // NSUB

BN = 2000


def _dot_wt(h, w):
    return lax.dot_general(h, w, (((1,), (1,)), ((), ())),
                           precision=lax.Precision.HIGHEST,
                           preferred_element_type=jnp.float32)


def _tc_first_body(temp_ref, x_ref, w_ref, b_ref, hh_ref, hid_ref):
    xb = x_ref[...]
    hh_ref[...] = _dot_wt(xb, w_ref[...]) + b_ref[...]
    hid_ref[...] = xb * temp_ref[0]


def _tc_mid_body(temp_ref, s_ref, hid_ref, w_ref, b_ref, hh_ref, hidout_ref,
                 *, layer):
    h = jnp.maximum(s_ref[0] + s_ref[1], 0.0)
    hidout_ref[...] = hid_ref[...] + h * temp_ref[layer]
    hh_ref[...] = _dot_wt(h, w_ref[...]) + b_ref[...]


def _tc_last_body(temp_ref, s_ref, hid_ref, out_ref):
    h = jnp.maximum(s_ref[0] + s_ref[1], 0.0)
    out_ref[...] = hid_ref[...] + h * temp_ref[3]


_SPEC_T = pl.BlockSpec(memory_space=pltpu.SMEM)
_SPEC_X = pl.BlockSpec((BN, D), lambda i: (i, 0))
_SPEC_S = pl.BlockSpec((2, BN, D), lambda i: (0, i, 0))
_SPEC_W = pl.BlockSpec((D, D), lambda i: (0, 0))
_SPEC_B = pl.BlockSpec((1, D), lambda i: (0, 0))

_GRID = N // BN


def _tc_first(temp, x, w, b):
    return pl.pallas_call(
        _tc_first_body,
        grid=(_GRID,),
        in_specs=[_SPEC_T, _SPEC_X, _SPEC_W, _SPEC_B],
        out_specs=[_SPEC_X, _SPEC_X],
        out_shape=[jax.ShapeDtypeStruct((NP, D), jnp.float32),
                   jax.ShapeDtypeStruct((N, D), jnp.float32)],
    )(temp, x, w, b)


def _tc_mid(temp, s, hid, w, b, layer):
    return pl.pallas_call(
        functools.partial(_tc_mid_body, layer=layer),
        grid=(_GRID,),
        in_specs=[_SPEC_T, _SPEC_S, _SPEC_X, _SPEC_W, _SPEC_B],
        out_specs=[_SPEC_X, _SPEC_X],
        out_shape=[jax.ShapeDtypeStruct((NP, D), jnp.float32),
                   jax.ShapeDtypeStruct((N, D), jnp.float32)],
    )(temp, s, hid, w, b)


def _tc_last(temp, s, hid):
    return pl.pallas_call(
        _tc_last_body,
        grid=(_GRID,),
        in_specs=[_SPEC_T, _SPEC_S, _SPEC_X],
        out_specs=_SPEC_X,
        out_shape=jax.ShapeDtypeStruct((N, D), jnp.float32),
    )(temp, s, hid)


def _sc_body(hh_hbm, src_hbm, dst_hbm, ew_hbm, out_hbm,
             acc_sh, src_v, dst_v, ew_v, rows_a, rows_b,
             gsem_a, gsem_b, ssem_a, ssem_b):
    c = lax.axis_index("c")
    s = lax.axis_index("s")
    w = c * NSUB + s

    def _zrow(r, carry):
        for k in range(D // 16):
            rows_a[r, pl.ds(k * 16, 16)] = jnp.zeros((16,), jnp.float32)
        return carry
    lax.fori_loop(0, ROWS_J, _zrow, 0)
    for z in range(RPT // ROWS_J):
        pltpu.sync_copy(rows_a, acc_sh.at[pl.ds(s * RPT + z * ROWS_J, ROWS_J)])

    plsc.subcore_barrier()

    def _scale(rows, ew_row):
        def _scale16(g, carry2):
            wv = ew_row[pl.ds(g * 16, 16)]
            for m in range(16):
                e = g * 16 + m
                ew = wv[m]
                for k in range(D // 16):
                    sl = (e, pl.ds(k * 16, 16))
                    rows[sl] = rows[sl] * ew
            return carry2
        lax.fori_loop(0, ROWS_J // 16, _scale16, 0)

    bufs = ((rows_a, gsem_a, ssem_a), (rows_b, gsem_b, ssem_b))

    def _gather(j, buf, gsem):
        pltpu.async_copy(hh_hbm.at[src_v.at[j]], buf, gsem)

    def _superchunk(t, carry):
        pltpu.sync_copy(src_hbm.at[w, pl.ds(t * NJB, NJB)], src_v)
        pltpu.sync_copy(dst_hbm.at[w, pl.ds(t * NJB, NJB)], dst_v)
        pltpu.sync_copy(ew_hbm.at[w, pl.ds(t * NJB, NJB)], ew_v)

        _gather(0, rows_a, gsem_a)
        _gather(1, rows_b, gsem_b)

        def _pair(i, carry1):
            for b, (buf, gsem, ssem) in enumerate(bufs):
                j = 2 * i + b
                pltpu.make_async_copy(hh_hbm.at[src_v.at[j]], buf, gsem).wait()
                _scale(buf, ew_v.at[j])
                pltpu.async_copy(buf, acc_sh.at[dst_v.at[j]], ssem, add=True)

                @pl.when(i < NJB // 2 - 1)
                def _prefetch():
                    pltpu.make_async_copy(
                        buf, acc_sh.at[dst_v.at[j]], ssem).wait()
                    _gather(j + 2, buf, gsem)
            return carry1
        lax.fori_loop(0, NJB // 2, _pair, 0)

        for b, (buf, gsem, ssem) in enumerate(bufs):
            j = NJB - 2 + b
            pltpu.make_async_copy(buf, acc_sh.at[dst_v.at[j]], ssem).wait()
        return carry
    lax.fori_loop(0, NT, _superchunk, 0)

    plsc.subcore_barrier()

    pltpu.sync_copy(acc_sh.at[pl.ds(s * RPT, RPT)],
                    out_hbm.at[c, pl.ds(s * RPT, RPT)])


@functools.lru_cache(maxsize=1)
def _make_sc_propagate():
    return functools.partial(
        pl.kernel,
        out_type=jax.ShapeDtypeStruct((2, NP, D), jnp.float32),
        mesh=plsc.VectorSubcoreMesh(core_axis_name="c", subcore_axis_name="s",
                                    num_cores=2, num_subcores=NSUB),
        scratch_types=[
            pltpu.VMEM_SHARED((NP, D), jnp.float32),
            pltpu.VMEM((NJB, ROWS_J), jnp.int32),
            pltpu.VMEM((NJB, ROWS_J), jnp.int32),
            pltpu.VMEM((NJB, ROWS_J), jnp.float32),
            pltpu.VMEM((ROWS_J, D), jnp.float32),
            pltpu.VMEM((ROWS_J, D), jnp.float32),
            pltpu.SemaphoreType.DMA,
            pltpu.SemaphoreType.DMA,
            pltpu.SemaphoreType.DMA,
            pltpu.SemaphoreType.DMA,
        ],
    )(_sc_body)


def _sc_propagate(*args):
    return _make_sc_propagate()(*args)


def kernel(x, edge_index, edge_weight, temp, W0, b0, W1, b1, W2, b2):
    pad = EPAD - E
    src = jnp.pad(edge_index[0].astype(jnp.int32), (0, pad))
    dst = jnp.pad(edge_index[1].astype(jnp.int32), (0, pad))
    ew = jnp.pad(edge_weight, (0, pad))
    src_t = src.reshape(NTILE, NJ, ROWS_J)
    dst_t = dst.reshape(NTILE, NJ, ROWS_J)
    ew_t = ew.reshape(NTILE, NJ, ROWS_J)

    hh, hidden = _tc_first(temp, x, W0, b0.reshape(1, D))
    s1 = _sc_propagate(hh, src_t, dst_t, ew_t)
    hh, hidden = _tc_mid(temp, s1, hidden, W1, b1.reshape(1, D), 1)
    s2 = _sc_propagate(hh, src_t, dst_t, ew_t)
    hh, hidden = _tc_mid(temp, s2, hidden, W2, b2.reshape(1, D), 2)
    s3 = _sc_propagate(hh, src_t, dst_t, ew_t)
    return _tc_last(temp, s3, hidden)
